# baseline (speedup 1.0000x reference)
import jax
import jax.numpy as jnp
from jax.experimental import pallas as pl
from jax.experimental.pallas import tpu as pltpu

N = 10000
M = 2048
B0 = 400
B1 = 400
T0 = N // B0
T1 = N // B1
EPS = 1e-9


def _kernel(h_ref, x_ref, z_ref, w_ref, wrow_ref,
            psi_W_ref, psi_b_ref, phi_W_ref, phi_b_ref,
            g1_W_ref, g1_b_ref, g2_W_ref, g2_b_ref,
            c1_W_ref, c1_b_ref, c2_W_ref, c2_b_ref,
            hd_W_ref, hd_b_ref,
            g_ref, out_ref,
            hq_scr, de_scr, m1t_scr, m2t_scr, mn_scr):
    t = pl.program_id(0)

    @pl.when(t == 0)
    def _init():
        de_scr[...] = jnp.zeros_like(de_scr)
        m1t_scr[...] = jnp.zeros_like(m1t_scr)
        m2t_scr[...] = jnp.zeros_like(m2t_scr)

    @pl.when(t < T0)
    def _phase0():
        h16 = h_ref[...].astype(jnp.bfloat16)
        hq_scr[pl.ds(t * B0, B0), :] = h16

        dv = jnp.dot(h16, w_ref[...],
                     preferred_element_type=jnp.float32)
        s = jax.lax.rsqrt(dv + EPS)
        ones = jnp.ones((1, B0), jnp.bfloat16)
        de_scr[...] += jnp.dot(ones, h16,
                               preferred_element_type=jnp.float32)

        x1 = x_ref[...] @ psi_W_ref[...] + psi_b_ref[...]
        z1 = z_ref[...] @ phi_W_ref[...] + phi_b_ref[...]
        cat = jnp.concatenate([x1, z1], axis=1)
        gh = jnp.maximum(cat @ g1_W_ref[...] + g1_b_ref[...], 0.0)
        g = jax.nn.sigmoid(gh @ g2_W_ref[...] + g2_b_ref[...])
        g_ref[...] = g
        fused = g * z1 + (1.0 - g) * x1

        x1c = fused @ c1_W_ref[...] + c1_b_ref[...]
        xn1 = (x1c * s).astype(jnp.bfloat16)
        m1t_scr[...] += jax.lax.dot_general(
            xn1, h16, (((0,), (0,)), ((), ())),
            preferred_element_type=jnp.float32)

    @pl.when(t == T0)
    def _norm1():
        se = wrow_ref[...] / (de_scr[...] + EPS)
        mn_scr[...] = jnp.transpose(
            (m1t_scr[...] * se).astype(jnp.bfloat16))

    @pl.when((t >= T0) & (t < T0 + T1))
    def _phase1():
        i = t - T0
        h16 = hq_scr[pl.ds(i * B1, B1), :]
        dv = jnp.dot(h16, w_ref[...],
                     preferred_element_type=jnp.float32)
        s = jax.lax.rsqrt(dv + EPS)
        y1 = jnp.dot(h16, mn_scr[...],
                     preferred_element_type=jnp.float32)
        h1 = jnp.maximum(y1 * s, 0.0)
        x2 = h1 @ c2_W_ref[...] + c2_b_ref[...]
        xn2 = (x2 * s).astype(jnp.bfloat16)
        m2t_scr[...] += jax.lax.dot_general(
            xn2, h16, (((0,), (0,)), ((), ())),
            preferred_element_type=jnp.float32)

    @pl.when(t == T0 + T1)
    def _norm2():
        se = wrow_ref[...] / (de_scr[...] + EPS)
        mn_scr[...] = jnp.transpose(
            (m2t_scr[...] * se).astype(jnp.bfloat16))

    @pl.when(t >= T0 + T1)
    def _phase2():
        i = t - (T0 + T1)
        h16 = hq_scr[pl.ds(i * B1, B1), :]
        dv = jnp.dot(h16, w_ref[...],
                     preferred_element_type=jnp.float32)
        s = jax.lax.rsqrt(dv + EPS)
        y2 = jnp.dot(h16, mn_scr[...],
                     preferred_element_type=jnp.float32)
        h2 = jnp.maximum(y2 * s, 0.0)
        out_ref[...] = h2 @ hd_W_ref[...] + hd_b_ref[...]


def _full(shape):
    nd = len(shape)
    return pl.BlockSpec(shape, lambda t: (0,) * nd)


def kernel(x, z, H, w,
           psi_W, psi_b, phi_W, phi_b,
           g1_W, g1_b, g2_W, g2_b,
           c1_W, c1_b, c2_W, c2_b,
           hd_W, hd_b):
    pin0 = lambda shape: pl.BlockSpec(
        shape, lambda t: (jnp.minimum(t, T0 - 1), 0))
    pin2 = lambda shape: pl.BlockSpec(
        shape, lambda t: (jnp.maximum(t - (T0 + T1), 0), 0))

    g, logits = pl.pallas_call(
        _kernel,
        grid=(T0 + 2 * T1,),
        in_specs=[pin0((B0, M)), pin0((B0, x.shape[1])),
                  pin0((B0, z.shape[1])),
                  _full((M, 1)), _full((1, M)),
                  _full(psi_W.shape), _full((1, psi_b.shape[0])),
                  _full(phi_W.shape), _full((1, phi_b.shape[0])),
                  _full(g1_W.shape), _full((1, g1_b.shape[0])),
                  _full(g2_W.shape), _full((1, g2_b.shape[0])),
                  _full(c1_W.shape), _full((1, c1_b.shape[0])),
                  _full(c2_W.shape), _full((1, c2_b.shape[0])),
                  _full(hd_W.shape), _full((1, hd_b.shape[0]))],
        out_specs=[pin0((B0, 32)), pin2((B1, hd_b.shape[0]))],
        out_shape=[jax.ShapeDtypeStruct((N, 32), jnp.float32),
                   jax.ShapeDtypeStruct((N, hd_b.shape[0]), jnp.float32)],
        scratch_shapes=[pltpu.VMEM((N, M), jnp.bfloat16),
                        pltpu.VMEM((1, M), jnp.float32),
                        pltpu.VMEM((64, M), jnp.float32),
                        pltpu.VMEM((64, M), jnp.float32),
                        pltpu.VMEM((M, 64), jnp.bfloat16)],
        compiler_params=pltpu.CompilerParams(
            dimension_semantics=("arbitrary",)),
    )(H, x, z, w.reshape(M, 1).astype(jnp.bfloat16), w.reshape(1, M),
      psi_W, psi_b.reshape(1, -1), phi_W, phi_b.reshape(1, -1),
      g1_W, g1_b.reshape(1, -1), g2_W, g2_b.reshape(1, -1),
      c1_W, c1_b.reshape(1, -1), c2_W, c2_b.reshape(1, -1),
      hd_W, hd_b.reshape(1, -1))

    return (logits, g)


# speedup vs baseline: 1.1815x; 1.1815x over previous
import jax
import jax.numpy as jnp
from jax.experimental import pallas as pl
from jax.experimental.pallas import tpu as pltpu

N = 10000
M = 2048
B0 = 400
B1 = 2000
T0 = N // B0
T1 = N // B1
T2 = N // B0
EPS = 1e-9


def _kernel(h_ref, x_ref, z_ref, w_ref, wrow_ref,
            psi_W_ref, psi_b_ref, phi_W_ref, phi_b_ref,
            g1_W_ref, g1_b_ref, g2_W_ref, g2_b_ref,
            c1_W_ref, c1_b_ref, c2_W_ref, c2_b_ref,
            hd_W_ref, hd_b_ref,
            g_ref, out_ref,
            hq_scr, de_scr, mt_scr, mn_scr):
    t = pl.program_id(0)

    @pl.when(t == 0)
    def _init():
        de_scr[...] = jnp.zeros_like(de_scr)
        mt_scr[...] = jnp.zeros_like(mt_scr)

    @pl.when(t < T0)
    def _phase0():
        h16 = h_ref[...].astype(jnp.bfloat16)

        dv = jnp.dot(h16, w_ref[...],
                     preferred_element_type=jnp.float32)
        s = jax.lax.rsqrt(dv + EPS)
        ones = jnp.ones((1, B0), jnp.bfloat16)
        de_scr[...] += jnp.dot(ones, h16,
                               preferred_element_type=jnp.float32)

        hs = h16 * s.astype(jnp.bfloat16)
        hq_scr[pl.ds(t * B0, B0), :] = hs

        x1 = x_ref[...] @ psi_W_ref[...] + psi_b_ref[...]
        z1 = z_ref[...] @ phi_W_ref[...] + phi_b_ref[...]
        cat = jnp.concatenate([x1, z1], axis=1)
        gh = jnp.maximum(cat @ g1_W_ref[...] + g1_b_ref[...], 0.0)
        g = jax.nn.sigmoid(gh @ g2_W_ref[...] + g2_b_ref[...])
        g_ref[...] = g
        fused = g * z1 + (1.0 - g) * x1

        x1c = (fused @ c1_W_ref[...] + c1_b_ref[...]
               ).astype(jnp.bfloat16)
        mt_scr[...] += jax.lax.dot_general(
            x1c, hs, (((0,), (0,)), ((), ())),
            preferred_element_type=jnp.float32)

    @pl.when(t == T0)
    def _norm1():
        se = wrow_ref[...] / (de_scr[...] + EPS)
        mn_scr[...] = jnp.transpose(
            (mt_scr[...] * se).astype(jnp.bfloat16))
        mt_scr[...] = jnp.zeros_like(mt_scr)

    @pl.when((t >= T0) & (t < T0 + T1))
    def _phase1():
        i = t - T0
        hs = hq_scr[pl.ds(i * B1, B1), :]
        y1 = jnp.dot(hs, mn_scr[...],
                     preferred_element_type=jnp.float32)
        h1 = jnp.maximum(y1, 0.0)
        x2 = (h1 @ c2_W_ref[...] + c2_b_ref[...]
              ).astype(jnp.bfloat16)
        mt_scr[...] += jax.lax.dot_general(
            x2, hs, (((0,), (0,)), ((), ())),
            preferred_element_type=jnp.float32)

    @pl.when(t == T0 + T1)
    def _norm2():
        se = wrow_ref[...] / (de_scr[...] + EPS)
        mn_scr[...] = jnp.transpose(
            (mt_scr[...] * se).astype(jnp.bfloat16))

    @pl.when(t >= T0 + T1)
    def _phase2():
        i = t - (T0 + T1)
        hs = hq_scr[pl.ds(i * B0, B0), :]
        y2 = jnp.dot(hs, mn_scr[...],
                     preferred_element_type=jnp.float32)
        h2 = jnp.maximum(y2, 0.0)
        out_ref[...] = h2 @ hd_W_ref[...] + hd_b_ref[...]


def _full(shape):
    nd = len(shape)
    return pl.BlockSpec(shape, lambda t: (0,) * nd)


def kernel(x, z, H, w,
           psi_W, psi_b, phi_W, phi_b,
           g1_W, g1_b, g2_W, g2_b,
           c1_W, c1_b, c2_W, c2_b,
           hd_W, hd_b):
    pin0 = lambda shape: pl.BlockSpec(
        shape, lambda t: (jnp.minimum(t, T0 - 1), 0))
    pin2 = lambda shape: pl.BlockSpec(
        shape, lambda t: (jnp.maximum(t - (T0 + T1), 0), 0))

    g, logits = pl.pallas_call(
        _kernel,
        grid=(T0 + T1 + T2,),
        in_specs=[pin0((B0, M)), pin0((B0, x.shape[1])),
                  pin0((B0, z.shape[1])),
                  _full((M, 1)), _full((1, M)),
                  _full(psi_W.shape), _full((1, psi_b.shape[0])),
                  _full(phi_W.shape), _full((1, phi_b.shape[0])),
                  _full(g1_W.shape), _full((1, g1_b.shape[0])),
                  _full(g2_W.shape), _full((1, g2_b.shape[0])),
                  _full(c1_W.shape), _full((1, c1_b.shape[0])),
                  _full(c2_W.shape), _full((1, c2_b.shape[0])),
                  _full(hd_W.shape), _full((1, hd_b.shape[0]))],
        out_specs=[pin0((B0, 32)), pin2((B0, hd_b.shape[0]))],
        out_shape=[jax.ShapeDtypeStruct((N, 32), jnp.float32),
                   jax.ShapeDtypeStruct((N, hd_b.shape[0]), jnp.float32)],
        scratch_shapes=[pltpu.VMEM((N, M), jnp.bfloat16),
                        pltpu.VMEM((1, M), jnp.float32),
                        pltpu.VMEM((64, M), jnp.float32),
                        pltpu.VMEM((M, 64), jnp.bfloat16)],
        compiler_params=pltpu.CompilerParams(
            dimension_semantics=("arbitrary",)),
    )(H, x, z, w.reshape(M, 1).astype(jnp.bfloat16), w.reshape(1, M),
      psi_W, psi_b.reshape(1, -1), phi_W, phi_b.reshape(1, -1),
      g1_W, g1_b.reshape(1, -1), g2_W, g2_b.reshape(1, -1),
      c1_W, c1_b.reshape(1, -1), c2_W, c2_b.reshape(1, -1),
      hd_W, hd_b.reshape(1, -1))

    return (logits, g)


# speedup vs baseline: 1.3580x; 1.1495x over previous
import jax
import jax.numpy as jnp
from jax.experimental import pallas as pl
from jax.experimental.pallas import tpu as pltpu

N = 10000
M = 2048
B = 1000
BP = 1024
T0 = N // B
T1 = 5
T2 = T0
NP = T0 * BP
EPS = 1e-9
ZPAD = BP - B
QS = 127.0


def _kernel(h_ref, x_ref, z_ref, w_ref, wrow_ref,
            psi_W_ref, psi_b_ref, phi_W_ref, phi_b_ref,
            g1_W_ref, g1_b_ref, g2_W_ref, g2_b_ref,
            c1_W_ref, c1_b_ref, c2_W_ref, c2_b_ref,
            hd_W_ref, hd_b_ref,
            g_ref, out_ref,
            hq_scr, s_scr, de_scr, mt_scr, mn_scr):
    t = pl.program_id(0)

    @pl.when(t == 0)
    def _init():
        de_scr[...] = jnp.zeros_like(de_scr)
        mt_scr[...] = jnp.zeros_like(mt_scr)

    @pl.when(t < T0)
    def _phase0():
        hf = h_ref[...]
        h16 = hf.astype(jnp.bfloat16)

        dv = jnp.dot(h16, w_ref[...],
                     preferred_element_type=jnp.float32)
        s = jax.lax.rsqrt(dv + EPS)
        s_scr[pl.ds(16 * t, 1), 0:B] = jnp.transpose(
            s.astype(jnp.bfloat16))
        ones = jnp.ones((1, B), jnp.bfloat16)
        de_scr[...] += jnp.dot(ones, h16,
                               preferred_element_type=jnp.float32)

        hq = (hf * QS + 0.5).astype(jnp.int8)
        hq_pad = jnp.concatenate(
            [hq, jnp.zeros((ZPAD, M), jnp.int8)], axis=0)
        hq_scr[pl.ds(t * BP, BP), :] = hq_pad

        x1 = x_ref[...] @ psi_W_ref[...] + psi_b_ref[...]
        z1 = z_ref[...] @ phi_W_ref[...] + phi_b_ref[...]
        cat = jnp.concatenate([x1, z1], axis=1)
        gh = jnp.maximum(cat @ g1_W_ref[...] + g1_b_ref[...], 0.0)
        g = jax.nn.sigmoid(gh @ g2_W_ref[...] + g2_b_ref[...])
        g_ref[...] = g
        fused = g * z1 + (1.0 - g) * x1

        x1c = fused @ c1_W_ref[...] + c1_b_ref[...]
        xn1 = (x1c * s).astype(jnp.bfloat16)
        mt_scr[...] += jax.lax.dot_general(
            xn1, h16, (((0,), (0,)), ((), ())),
            preferred_element_type=jnp.float32)

    @pl.when(t == T0)
    def _norm1():
        se = wrow_ref[...] / (de_scr[...] + EPS)
        mn_scr[...] = jnp.transpose(
            (mt_scr[...] * se).astype(jnp.bfloat16))
        mt_scr[...] = jnp.zeros_like(mt_scr)

    @pl.when((t >= T0) & (t < T0 + T1))
    def _phase1():
        i = t - T0
        hs = hq_scr[pl.ds(i * (2 * BP), 2 * BP), :
                    ].astype(jnp.bfloat16)
        sa = jnp.transpose(s_scr[pl.ds(32 * i, 1), 0:B])
        sb = jnp.transpose(s_scr[pl.ds(32 * i + 16, 1), 0:B])
        zp = jnp.zeros((ZPAD, 1), jnp.bfloat16)
        sc = jnp.concatenate([sa, zp, sb, zp], axis=0
                             ).astype(jnp.float32)
        y1 = jnp.dot(hs, mn_scr[...],
                     preferred_element_type=jnp.float32) * (1.0 / QS)
        h1 = jnp.maximum(y1 * sc, 0.0)
        x2 = h1 @ c2_W_ref[...] + c2_b_ref[...]
        xn2 = (x2 * sc).astype(jnp.bfloat16)
        mt_scr[...] += jax.lax.dot_general(
            xn2, hs, (((0,), (0,)), ((), ())),
            preferred_element_type=jnp.float32)

    @pl.when(t == T0 + T1)
    def _norm2():
        se = wrow_ref[...] / ((de_scr[...] + EPS) * QS)
        mn_scr[...] = jnp.transpose(
            (mt_scr[...] * se).astype(jnp.bfloat16))

    @pl.when(t >= T0 + T1)
    def _phase2():
        i = t - (T0 + T1)
        hs = hq_scr[pl.ds(i * BP, BP), :].astype(jnp.bfloat16)
        sc = jnp.transpose(s_scr[pl.ds(16 * i, 1), 0:B]
                           ).astype(jnp.float32)
        y2 = jnp.dot(hs, mn_scr[...],
                     preferred_element_type=jnp.float32) * (1.0 / QS)
        h2 = jnp.maximum(y2[0:B] * sc, 0.0)
        out_ref[...] = h2 @ hd_W_ref[...] + hd_b_ref[...]


def _full(shape):
    nd = len(shape)
    return pl.BlockSpec(shape, lambda t: (0,) * nd)


def kernel(x, z, H, w,
           psi_W, psi_b, phi_W, phi_b,
           g1_W, g1_b, g2_W, g2_b,
           c1_W, c1_b, c2_W, c2_b,
           hd_W, hd_b):
    rmap = lambda t: (jnp.minimum(t, T0 - 1), 0)
    pin2 = lambda shape: pl.BlockSpec(
        shape, lambda t: (jnp.maximum(t - (T0 + T1), 0), 0))

    g, logits = pl.pallas_call(
        _kernel,
        grid=(T0 + T1 + T2,),
        in_specs=[pl.BlockSpec((B, M), rmap),
                  pl.BlockSpec((B, x.shape[1]), rmap),
                  pl.BlockSpec((B, z.shape[1]), rmap),
                  _full((M, 1)), _full((1, M)),
                  _full(psi_W.shape), _full((1, psi_b.shape[0])),
                  _full(phi_W.shape), _full((1, phi_b.shape[0])),
                  _full(g1_W.shape), _full((1, g1_b.shape[0])),
                  _full(g2_W.shape), _full((1, g2_b.shape[0])),
                  _full(c1_W.shape), _full((1, c1_b.shape[0])),
                  _full(c2_W.shape), _full((1, c2_b.shape[0])),
                  _full(hd_W.shape), _full((1, hd_b.shape[0]))],
        out_specs=[pl.BlockSpec((B, 32), rmap),
                   pin2((B, hd_b.shape[0]))],
        out_shape=[jax.ShapeDtypeStruct((N, 32), jnp.float32),
                   jax.ShapeDtypeStruct((N, hd_b.shape[0]), jnp.float32)],
        scratch_shapes=[pltpu.VMEM((NP, M), jnp.int8),
                        pltpu.VMEM((16 * T0, 1024), jnp.bfloat16),
                        pltpu.VMEM((1, M), jnp.float32),
                        pltpu.VMEM((64, M), jnp.float32),
                        pltpu.VMEM((M, 64), jnp.bfloat16)],
        compiler_params=pltpu.CompilerParams(
            dimension_semantics=("arbitrary",)),
    )(H, x, z, w.reshape(M, 1).astype(jnp.bfloat16), w.reshape(1, M),
      psi_W, psi_b.reshape(1, -1), phi_W, phi_b.reshape(1, -1),
      g1_W, g1_b.reshape(1, -1), g2_W, g2_b.reshape(1, -1),
      c1_W, c1_b.reshape(1, -1), c2_W, c2_b.reshape(1, -1),
      hd_W, hd_b.reshape(1, -1))

    return (logits, g)
